# baseline (speedup 1.0000x reference)
import jax
import jax.numpy as jnp
from jax.experimental import pallas as pl
from jax.experimental.pallas import tpu as pltpu

N_BINS = 100
BW = 0.01
B, H, W = 4, 512, 512
R = 8
G = (B * H) // R
BLOCKS_PER_BATCH = H // R


def _pdf(a):
    p = a * (1.0 / BW) - 0.5
    k = jax.lax.broadcasted_iota(jnp.int32, (1, 1, N_BINS), 2).astype(jnp.float32)
    tri = jnp.maximum(1.0 - jnp.abs(p[:, :, None] - k), 0.0)
    h = jnp.sum(tri, axis=1)
    return h / (jnp.sum(h, axis=-1, keepdims=True) + 1e-10)


def _loss_kernel(s1_ref, s2_ref, x_ref, t_ref, xc_ref, tc_ref, s3_ref, out_ref):
    i = pl.program_id(0)
    b = i // BLOCKS_PER_BATCH

    x = x_ref[...]
    t = t_ref[...]

    diffsum = jnp.sum(jnp.abs(x - t))

    pn = _pdf(x) + 1e-5
    pc = _pdf(t) + 1e-5
    kl = jnp.sum(pc * (jnp.log(pc) - jnp.log(pn)))

    out_ref[...] = (s1_ref[b] * diffsum + s2_ref[b] * kl).reshape(1, 1, 1)

    @pl.when(i == G - 1)
    def _():
        pnc = _pdf(xc_ref[...]) + 1e-6
        pcc = _pdf(tc_ref[...]) + 1e-6
        klc = jnp.sum(pcc * (jnp.log(pcc) - jnp.log(pnc)), axis=-1,
                      keepdims=True)
        colp = jnp.sum(s3_ref[...] * klc)
        out_ref[...] = out_ref[...] + colp


def kernel(inputo, target, we1, we2, we3):
    eps = 1e-6
    w1 = we1.reshape(B) + eps
    w2 = we2.reshape(B) + eps
    w3 = we3.reshape(B) + eps
    n_total = B * H * W
    s1 = (w1 + 1.0 / w1) / n_total
    s2 = (w2 + 1.0 / w2) / (2 * B * H)
    s3 = ((w3 + 1.0 / w3) / (2 * B * H)).reshape(B, 1)

    x = inputo.reshape(B * H, W)
    t = target.reshape(B * H, W)
    xc = inputo[:, 0, :, 0]
    tc = target[:, 0, :, 0]

    partials = pl.pallas_call(
        _loss_kernel,
        out_shape=jax.ShapeDtypeStruct((G, 1, 1), jnp.float32),
        grid=(G,),
        in_specs=[
            pl.BlockSpec(memory_space=pltpu.SMEM),
            pl.BlockSpec(memory_space=pltpu.SMEM),
            pl.BlockSpec((R, W), lambda i: (i, 0)),
            pl.BlockSpec((R, W), lambda i: (i, 0)),
            pl.BlockSpec((B, H), lambda i: (0, 0)),
            pl.BlockSpec((B, H), lambda i: (0, 0)),
            pl.BlockSpec((B, 1), lambda i: (0, 0)),
        ],
        out_specs=pl.BlockSpec((1, 1, 1), lambda i: (i, 0, 0)),
        compiler_params=pltpu.CompilerParams(
            dimension_semantics=("parallel",),
        ),
        name="customloss_kll",
    )(s1, s2, x, t, xc, tc, s3)

    return jnp.sum(partials)


# speedup vs baseline: 1.8213x; 1.8213x over previous
import jax
import jax.numpy as jnp
from jax.experimental import pallas as pl
from jax.experimental.pallas import tpu as pltpu

N_BINS = 100
BW = 0.01
B, H, W = 4, 512, 512
R = 8
G = (B * H) // R
BLOCKS_PER_BATCH = H // R


N_KNOTS = N_BINS + 2


def _psi(a, s_ref):
    rows = a.shape[0]
    p1 = a * (1.0 / BW) + 0.5
    j = jax.lax.broadcasted_iota(
        jnp.int32, (1, N_KNOTS, W), 1).astype(jnp.float32)
    ramp = jnp.maximum(p1[:, None, :] - j, 0.0)
    folded = (ramp[:, :, 0:128] + ramp[:, :, 128:256]
              + ramp[:, :, 256:384] + ramp[:, :, 384:512])
    s_ref[0:rows, :] = jnp.sum(folded, axis=-1)


def _pdf(s):
    h = s[:, :N_BINS] - 2.0 * s[:, 1:N_BINS + 1] + s[:, 2:N_BINS + 2]
    return h / (jnp.sum(h, axis=-1, keepdims=True) + 1e-10)


def _loss_kernel(s1_ref, s2_ref, x_ref, t_ref, xc_ref, tc_ref, s3_ref, out_ref,
                 sx_ref, st_ref):
    i = pl.program_id(0)
    b = i // BLOCKS_PER_BATCH

    x = x_ref[...]
    t = t_ref[...]

    diffsum = jnp.sum(jnp.abs(x - t))

    _psi(x, sx_ref)
    _psi(t, st_ref)
    pn = _pdf(sx_ref[...]) + 1e-5
    pc = _pdf(st_ref[...]) + 1e-5
    kl = jnp.sum(pc * (jnp.log(pc) - jnp.log(pn)))

    out_ref[...] = (s1_ref[b] * diffsum + s2_ref[b] * kl).reshape(1, 1, 1)

    @pl.when(i == G - 1)
    def _():
        _psi(xc_ref[...], sx_ref)
        _psi(tc_ref[...], st_ref)
        pnc = _pdf(sx_ref[0:B, :]) + 1e-6
        pcc = _pdf(st_ref[0:B, :]) + 1e-6
        klc = jnp.sum(pcc * (jnp.log(pcc) - jnp.log(pnc)), axis=-1,
                      keepdims=True)
        colp = jnp.sum(s3_ref[...] * klc)
        out_ref[...] = out_ref[...] + colp


def kernel(inputo, target, we1, we2, we3):
    eps = 1e-6
    w1 = we1.reshape(B) + eps
    w2 = we2.reshape(B) + eps
    w3 = we3.reshape(B) + eps
    n_total = B * H * W
    s1 = (w1 + 1.0 / w1) / n_total
    s2 = (w2 + 1.0 / w2) / (2 * B * H)
    s3 = ((w3 + 1.0 / w3) / (2 * B * H)).reshape(B, 1)

    x = inputo.reshape(B * H, W)
    t = target.reshape(B * H, W)
    xc = inputo[:, 0, :, 0]
    tc = target[:, 0, :, 0]

    partials = pl.pallas_call(
        _loss_kernel,
        out_shape=jax.ShapeDtypeStruct((G, 1, 1), jnp.float32),
        grid=(G,),
        in_specs=[
            pl.BlockSpec(memory_space=pltpu.SMEM),
            pl.BlockSpec(memory_space=pltpu.SMEM),
            pl.BlockSpec((R, W), lambda i: (i, 0)),
            pl.BlockSpec((R, W), lambda i: (i, 0)),
            pl.BlockSpec((B, H), lambda i: (0, 0)),
            pl.BlockSpec((B, H), lambda i: (0, 0)),
            pl.BlockSpec((B, 1), lambda i: (0, 0)),
        ],
        out_specs=pl.BlockSpec((1, 1, 1), lambda i: (i, 0, 0)),
        scratch_shapes=[
            pltpu.VMEM((R, N_KNOTS), jnp.float32),
            pltpu.VMEM((R, N_KNOTS), jnp.float32),
        ],
        compiler_params=pltpu.CompilerParams(
            dimension_semantics=("parallel",),
        ),
        name="customloss_kll",
    )(s1, s2, x, t, xc, tc, s3)

    return jnp.sum(partials)


# speedup vs baseline: 3.2173x; 1.7665x over previous
import jax
import jax.numpy as jnp
from jax.experimental import pallas as pl
from jax.experimental.pallas import tpu as pltpu

N_BINS = 100
BW = 0.01
B, H, W = 4, 512, 512
R = 64
G = (B * H) // R
BLOCKS_PER_BATCH = H // R


N_KNOTS = N_BINS + 2


def _psi(a, s_ref):
    rows = a.shape[0]
    p1 = a * (1.0 / BW) + 0.5
    j = jax.lax.broadcasted_iota(
        jnp.int32, (1, N_KNOTS, W), 1).astype(jnp.float32)
    ramp = jnp.maximum(p1[:, None, :] - j, 0.0)
    folded = (ramp[:, :, 0:128] + ramp[:, :, 128:256]
              + ramp[:, :, 256:384] + ramp[:, :, 384:512])
    s_ref[0:rows, :] = jnp.sum(folded, axis=-1)


def _pdf(s):
    h = s[:, :N_BINS] - 2.0 * s[:, 1:N_BINS + 1] + s[:, 2:N_BINS + 2]
    return h / (jnp.sum(h, axis=-1, keepdims=True) + 1e-10)


def _loss_kernel(s1_ref, s2_ref, x_ref, t_ref, xc_ref, tc_ref, s3_ref, out_ref,
                 sx_ref, st_ref):
    i = pl.program_id(0)
    b = i // BLOCKS_PER_BATCH

    x = x_ref[...]
    t = t_ref[...]

    diffsum = jnp.sum(jnp.abs(x - t))

    _psi(x, sx_ref)
    _psi(t, st_ref)
    pn = _pdf(sx_ref[...]) + 1e-5
    pc = _pdf(st_ref[...]) + 1e-5
    kl = jnp.sum(pc * (jnp.log(pc) - jnp.log(pn)))

    out_ref[...] = (s1_ref[b] * diffsum + s2_ref[b] * kl).reshape(1, 1, 1)

    @pl.when(i == G - 1)
    def _():
        _psi(xc_ref[...], sx_ref)
        _psi(tc_ref[...], st_ref)
        pnc = _pdf(sx_ref[0:B, :]) + 1e-6
        pcc = _pdf(st_ref[0:B, :]) + 1e-6
        klc = jnp.sum(pcc * (jnp.log(pcc) - jnp.log(pnc)), axis=-1,
                      keepdims=True)
        colp = jnp.sum(s3_ref[...] * klc)
        out_ref[...] = out_ref[...] + colp


def kernel(inputo, target, we1, we2, we3):
    eps = 1e-6
    w1 = we1.reshape(B) + eps
    w2 = we2.reshape(B) + eps
    w3 = we3.reshape(B) + eps
    n_total = B * H * W
    s1 = (w1 + 1.0 / w1) / n_total
    s2 = (w2 + 1.0 / w2) / (2 * B * H)
    s3 = ((w3 + 1.0 / w3) / (2 * B * H)).reshape(B, 1)

    x = inputo.reshape(B * H, W)
    t = target.reshape(B * H, W)
    xc = inputo[:, 0, :, 0]
    tc = target[:, 0, :, 0]

    partials = pl.pallas_call(
        _loss_kernel,
        out_shape=jax.ShapeDtypeStruct((G, 1, 1), jnp.float32),
        grid=(G,),
        in_specs=[
            pl.BlockSpec(memory_space=pltpu.SMEM),
            pl.BlockSpec(memory_space=pltpu.SMEM),
            pl.BlockSpec((R, W), lambda i: (i, 0)),
            pl.BlockSpec((R, W), lambda i: (i, 0)),
            pl.BlockSpec((B, H), lambda i: (0, 0)),
            pl.BlockSpec((B, H), lambda i: (0, 0)),
            pl.BlockSpec((B, 1), lambda i: (0, 0)),
        ],
        out_specs=pl.BlockSpec((1, 1, 1), lambda i: (i, 0, 0)),
        scratch_shapes=[
            pltpu.VMEM((R, N_KNOTS), jnp.float32),
            pltpu.VMEM((R, N_KNOTS), jnp.float32),
        ],
        compiler_params=pltpu.CompilerParams(
            dimension_semantics=("parallel",),
        ),
        name="customloss_kll",
    )(s1, s2, x, t, xc, tc, s3)

    return jnp.sum(partials)


# speedup vs baseline: 3.4124x; 1.0606x over previous
import jax
import jax.numpy as jnp
from jax.experimental import pallas as pl
from jax.experimental.pallas import tpu as pltpu

N_BINS = 100
BW = 0.01
B, H, W = 4, 512, 512
R = 128
G = (B * H) // R
BLOCKS_PER_BATCH = H // R


N_KNOTS = N_BINS + 2


def _psi(a, s_ref):
    rows = a.shape[0]
    p1 = a * (1.0 / BW) + 0.5
    j = jax.lax.broadcasted_iota(
        jnp.int32, (1, N_KNOTS, W), 1).astype(jnp.float32)
    ramp = jnp.maximum(p1[:, None, :] - j, 0.0)
    folded = (ramp[:, :, 0:128] + ramp[:, :, 128:256]
              + ramp[:, :, 256:384] + ramp[:, :, 384:512])
    s_ref[0:rows, :] = jnp.sum(folded, axis=-1)


def _pdf(s):
    h = s[:, :N_BINS] - 2.0 * s[:, 1:N_BINS + 1] + s[:, 2:N_BINS + 2]
    return h / (jnp.sum(h, axis=-1, keepdims=True) + 1e-10)


def _loss_kernel(s1_ref, s2_ref, x_ref, t_ref, xc_ref, tc_ref, s3_ref, out_ref,
                 sx_ref, st_ref):
    i = pl.program_id(0)
    b = i // BLOCKS_PER_BATCH

    x = x_ref[...]
    t = t_ref[...]

    diffsum = jnp.sum(jnp.abs(x - t))

    _psi(x, sx_ref)
    _psi(t, st_ref)
    pn = _pdf(sx_ref[...]) + 1e-5
    pc = _pdf(st_ref[...]) + 1e-5
    kl = jnp.sum(pc * (jnp.log(pc) - jnp.log(pn)))

    out_ref[...] = (s1_ref[b] * diffsum + s2_ref[b] * kl).reshape(1, 1, 1)

    @pl.when(i == G - 1)
    def _():
        _psi(xc_ref[...], sx_ref)
        _psi(tc_ref[...], st_ref)
        pnc = _pdf(sx_ref[0:B, :]) + 1e-6
        pcc = _pdf(st_ref[0:B, :]) + 1e-6
        klc = jnp.sum(pcc * (jnp.log(pcc) - jnp.log(pnc)), axis=-1,
                      keepdims=True)
        colp = jnp.sum(s3_ref[...] * klc)
        out_ref[...] = out_ref[...] + colp


def kernel(inputo, target, we1, we2, we3):
    eps = 1e-6
    w1 = we1.reshape(B) + eps
    w2 = we2.reshape(B) + eps
    w3 = we3.reshape(B) + eps
    n_total = B * H * W
    s1 = (w1 + 1.0 / w1) / n_total
    s2 = (w2 + 1.0 / w2) / (2 * B * H)
    s3 = ((w3 + 1.0 / w3) / (2 * B * H)).reshape(B, 1)

    x = inputo.reshape(B * H, W)
    t = target.reshape(B * H, W)
    xc = inputo[:, 0, :, 0]
    tc = target[:, 0, :, 0]

    partials = pl.pallas_call(
        _loss_kernel,
        out_shape=jax.ShapeDtypeStruct((G, 1, 1), jnp.float32),
        grid=(G,),
        in_specs=[
            pl.BlockSpec(memory_space=pltpu.SMEM),
            pl.BlockSpec(memory_space=pltpu.SMEM),
            pl.BlockSpec((R, W), lambda i: (i, 0)),
            pl.BlockSpec((R, W), lambda i: (i, 0)),
            pl.BlockSpec((B, H), lambda i: (0, 0)),
            pl.BlockSpec((B, H), lambda i: (0, 0)),
            pl.BlockSpec((B, 1), lambda i: (0, 0)),
        ],
        out_specs=pl.BlockSpec((1, 1, 1), lambda i: (i, 0, 0)),
        scratch_shapes=[
            pltpu.VMEM((R, N_KNOTS), jnp.float32),
            pltpu.VMEM((R, N_KNOTS), jnp.float32),
        ],
        compiler_params=pltpu.CompilerParams(
            dimension_semantics=("arbitrary",),
        ),
        name="customloss_kll",
    )(s1, s2, x, t, xc, tc, s3)

    return jnp.sum(partials)
